# baseline (speedup 1.0000x reference)
import functools

import jax
import jax.numpy as jnp
from jax import lax
from jax.experimental import pallas as pl
from jax.experimental.pallas import tpu as pltpu
from jax.experimental.pallas import tpu_sc as plsc

B = 16384
D = 786
DP = 792
NC, NS = 2, 16
NW = NC * NS
B_PER_W = B // NW
CHUNK = 64
NCHUNK = B_PER_W // CHUNK


def _make_sc_gather():
    mesh = plsc.VectorSubcoreMesh(core_axis_name="c", subcore_axis_name="s")

    @functools.partial(
        pl.kernel,
        mesh=mesh,
        out_type=jax.ShapeDtypeStruct((B, DP), jnp.float32),
        scratch_types=[
            pltpu.VMEM((NCHUNK, CHUNK), jnp.int32),
            pltpu.VMEM((2, CHUNK, DP), jnp.float32),
            pltpu.SemaphoreType.DMA,
        ],
        compiler_params=pltpu.CompilerParams(use_tc_tiling_on_sc=False),
    )
    def run(t_hbm, table_hbm, out_hbm, idx_v, rows_v, gsem):
        wid = lax.axis_index("s") * NC + lax.axis_index("c")
        base = wid * B_PER_W
        pltpu.sync_copy(t_hbm.at[wid], idx_v)
        for c in range(NCHUNK):
            buf = c % 2
            pltpu.async_copy(
                table_hbm.at[idx_v.at[c]], rows_v.at[buf], gsem
            ).wait()
            pltpu.sync_copy(
                rows_v.at[buf], out_hbm.at[pl.ds(base + c * CHUNK, CHUNK)]
            )

    return run


_sc_gather = _make_sc_gather()


def kernel(t, sinusoidal_embeddings):
    t_r = t.astype(jnp.int32).reshape(NW, NCHUNK, CHUNK)
    table_p = jnp.pad(sinusoidal_embeddings, ((0, 0), (0, DP - D)))
    out_p = _sc_gather(t_r, table_p)
    return out_p[:, :D]
